# baseline (speedup 1.0000x reference)
import jax
import jax.numpy as jnp
from jax.experimental import pallas as pl
from jax.experimental.pallas import tpu as pltpu

_K = 1024
_D = 64
_R = 4608


def _vq_block(x_ref, cb_ref, out_ref):
    x = x_ref[...]
    cb = cb_ref[...]
    c2 = jnp.sum(cb * cb, axis=1)
    v2 = jnp.sum(x * x, axis=1, keepdims=True)
    cross2 = jax.lax.dot_general(
        x, cb + cb, (((1,), (1,)), ((), ())),
        preferred_element_type=jnp.float32)
    dist = v2 - cross2 + c2[None, :]
    tok = jnp.argmin(dist, axis=1).astype(jnp.int32)
    iota = jax.lax.broadcasted_iota(jnp.int32, (_R, _K), 1)
    onehot = (iota == tok[:, None]).astype(jnp.float32)
    emb = jax.lax.dot_general(
        onehot, cb, (((1,), (0,)), ((), ())),
        preferred_element_type=jnp.float32)
    out_ref[...] = x + (emb - x)


def kernel(inputs, codebook, training):
    del training
    b, h, w, d = inputs.shape
    n = b * h * w
    x = inputs.reshape(n, d)
    out = pl.pallas_call(
        _vq_block,
        grid=(n // _R,),
        in_specs=[
            pl.BlockSpec((_R, d), lambda i: (i, 0)),
            pl.BlockSpec((_K, d), lambda i: (0, 0)),
        ],
        out_specs=pl.BlockSpec((_R, d), lambda i: (i, 0)),
        out_shape=jax.ShapeDtypeStruct((n, d), jnp.float32),
    )(x, codebook)
    return out.reshape(b, h, w, d)
